# baseline (speedup 1.0000x reference)
import functools

import jax
import jax.numpy as jnp
from jax import lax
from jax.experimental import pallas as pl
from jax.experimental.pallas import tpu as pltpu
from jax.experimental.pallas import tpu_sc as plsc

_B = 16384
_F = 16
_NMCOL = 782
_NMPAD = _NMCOL * 128

_info = plsc.get_sparse_core_info()
_NC, _NS = _info.num_cores, _info.num_subcores
_NW = _NC * _NS
_BPW = _B // _NW
_MCPW = (_NMCOL + _NS - 1) // _NS

_mesh = plsc.VectorSubcoreMesh(core_axis_name="c", subcore_axis_name="s")


@functools.partial(
    pl.kernel,
    out_type=(jax.ShapeDtypeStruct((_B,), jnp.float32),
              jax.ShapeDtypeStruct((_NMCOL * 16, 128), jnp.float32)),
    mesh=_mesh,
    compiler_params=pltpu.CompilerParams(needs_layout_passes=False),
    scratch_types=[
        pltpu.VMEM((_BPW,), jnp.int32),
        pltpu.VMEM((_BPW,), jnp.int32),
        pltpu.VMEM((_BPW,), jnp.int32),
        pltpu.VMEM((2, 16, 2, 8, 128), jnp.float32),
        pltpu.VMEM((2, 16, 128), jnp.float32),
        pltpu.VMEM((2, 16, 128), jnp.float32),
        pltpu.VMEM((2, 16, 128), jnp.float32),
        pltpu.VMEM((_BPW,), jnp.float32),
        pltpu.SemaphoreType.DMA,
        pltpu.SemaphoreType.DMA,
        pltpu.SemaphoreType.DMA,
        pltpu.SemaphoreType.DMA,
        pltpu.SemaphoreType.DMA,
        pltpu.SemaphoreType.DMA,
    ],
)
def _sc_dot_kernel(user_h, movie_h, ut3_h, mtT_h, out_h, spm,
                   uidx, midx, mgidx, ublk, mbuf, bounce, trows, outv,
                   sem_u0, sem_m0, sem_u1, sem_m1, sem_st, sem_wb):
    sid = lax.axis_index("s")
    wid = sid * _NC + lax.axis_index("c")
    base = wid * _BPW

    pltpu.sync_copy(user_h.at[pl.ds(base, _BPW)], uidx)
    pltpu.sync_copy(movie_h.at[pl.ds(base, _BPW)], midx)

    lane = lax.iota(jnp.int32, 16)

    def mg_body(i, carry):
        mgidx[pl.ds(i * 16, 16)] = midx[pl.ds(i * 16, 16)] >> 3
        return carry

    lax.fori_loop(0, _BPW // 16, mg_body, 0, unroll=False)

    def col_of(k):
        return sid * _MCPW + k

    def st_fetch(k, kb):
        @pl.when(col_of(k) < _NMCOL)
        def _():
            j0 = pl.multiple_of(col_of(k) * 128, 128)
            pltpu.async_copy(mtT_h.at[:, pl.ds(j0, 128)],
                             bounce.at[kb], sem_st)

    st_fetch(0, 0)

    def st_body(k, carry):
        kb = k & 1
        st_fetch(k + 1, 1 - kb)

        @pl.when(col_of(k) < _NMCOL)
        def _():
            pltpu.make_async_copy(mtT_h.at[:, pl.ds(0, 128)],
                                  bounce.at[0], sem_st).wait()
            kbv = jnp.zeros((16,), jnp.int32) + kb

            def tr(c, carry2):
                cv = jnp.zeros((16,), jnp.int32) + c
                row = plsc.load_gather(bounce, [kbv, lane, cv])
                trows[kb, c >> 3, pl.ds((c & 7) * 16, 16)] = row
                return carry2

            lax.fori_loop(0, 128, tr, 0, unroll=False)

        @pl.when(jnp.logical_and(k > 0, col_of(k - 1) < _NMCOL))
        def _():
            pltpu.make_async_copy(trows.at[0],
                                  spm.at[pl.ds(0, 16), :], sem_wb).wait()

        @pl.when(col_of(k) < _NMCOL)
        def _():
            q0 = pl.multiple_of(col_of(k) * 16, 16)
            pltpu.async_copy(trows.at[kb],
                             spm.at[pl.ds(q0, 16), :], sem_wb)

        return carry

    lax.fori_loop(0, _MCPW, st_body, 0, unroll=False)

    @pl.when(col_of(_MCPW - 1) < _NMCOL)
    def _():
        pltpu.make_async_copy(trows.at[0],
                              spm.at[pl.ds(0, 16), :], sem_wb).wait()

    plsc.subcore_barrier()

    def issue_chunk(c, buf, sem_u, sem_m):
        b0 = c * 16
        u16 = uidx[pl.ds(b0, 16)]
        for r in range(16):
            j0 = pl.multiple_of((u16[r] >> 7) * 128, 128)
            pltpu.async_copy(ut3_h.at[:, :, pl.ds(j0, 128)],
                             ublk.at[buf, r], sem_u)
        pltpu.async_copy(spm.at[mgidx.at[pl.ds(b0, 16)]],
                         mbuf.at[buf], sem_m)

    def wait_chunk(sem_u, sem_m):
        for _ in range(16):
            pltpu.make_async_copy(ut3_h.at[:, :, pl.ds(0, 128)],
                                  ublk.at[0, 0], sem_u).wait()
        pltpu.make_async_copy(spm.at[mgidx.at[pl.ds(0, 16)]],
                              mbuf.at[0], sem_m).wait()

    def compute_chunk(c, buf):
        b0 = c * 16
        u16 = uidx[pl.ds(b0, 16)]
        m16 = midx[pl.ds(b0, 16)]
        cv = u16 & 127
        mcol0 = (m16 & 7) * 16
        bufv = jnp.zeros((16,), jnp.int32) + buf
        acc = jnp.zeros((16,), jnp.float32)
        for f in range(_F):
            hiv = jnp.full((16,), f >> 3, jnp.int32)
            lov = jnp.full((16,), f & 7, jnp.int32)
            uf = plsc.load_gather(ublk, [bufv, lane, hiv, lov, cv])
            mf = plsc.load_gather(mbuf, [bufv, lane, mcol0 + f])
            acc = acc + uf * mf
        outv[pl.ds(b0, 16)] = acc

    n_pairs = _BPW // 32
    issue_chunk(0, 0, sem_u0, sem_m0)

    def u_body(k, carry):
        c = k * 2
        issue_chunk(c + 1, 1, sem_u1, sem_m1)
        wait_chunk(sem_u0, sem_m0)
        compute_chunk(c, 0)

        @pl.when(k < n_pairs - 1)
        def _():
            issue_chunk(c + 2, 0, sem_u0, sem_m0)

        wait_chunk(sem_u1, sem_m1)
        compute_chunk(c + 1, 1)
        return carry

    lax.fori_loop(0, n_pairs, u_body, 0, unroll=False)

    pltpu.sync_copy(outv, out_h.at[pl.ds(base, _BPW)])


def kernel(user, movie, user_to_feature, movie_to_feature):
    n_u = user_to_feature.shape[0]
    ut3 = user_to_feature.T.reshape(2, 8, n_u)
    mtT = movie_to_feature.T
    out, _ = _sc_dot_kernel(user, movie, ut3, mtT)
    return out
